# Initial kernel scaffold; baseline (speedup 1.0000x reference)
import jax
import jax.numpy as jnp
from jax.experimental import pallas as pl


def kernel(inputs, targets, kernel):
    raise NotImplementedError("write your pallas kernel here")


# speedup vs baseline: 1.5627x; 1.5627x over previous
import functools

import jax
import jax.numpy as jnp
from jax.experimental import pallas as pl
from jax.experimental.pallas import tpu as pltpu

_ALPHA = 10.0
_BETA = 2.0


def _fused_loss_kernel(x_tile_ref, x_full_ref, k_ref, t_col_ref, t_row_ref,
                       acc_ref, *, n_classes):
    i = pl.program_id(0)
    x_tile = x_tile_ref[...]
    x_full = x_full_ref[...]
    kmat = k_ref[...]
    t_col = t_col_ref[...]
    t_row = t_row_ref[...]

    sim = jax.lax.dot_general(x_tile, x_full, (((1,), (1,)), ((), ())),
                              preferred_element_type=jnp.float32)
    same = t_col == t_row
    pos_mask = jnp.logical_and(same, sim < 1.0)
    pos_cnt = jnp.sum(pos_mask.astype(jnp.float32), axis=1, keepdims=True)
    neg_cnt = jnp.sum(1.0 - same.astype(jnp.float32), axis=1, keepdims=True)
    pos_sum = jnp.sum(jnp.where(pos_mask, sim, 0.0), axis=1, keepdims=True)
    tot_sum = jnp.sum(sim, axis=1, keepdims=True)
    same_sum = jnp.sum(jnp.where(same, sim, 0.0), axis=1, keepdims=True)
    neg_sum = tot_sum - same_sum
    pos_part = jnp.sum(pos_sum / pos_cnt)
    neg_part = jnp.sum(neg_sum / neg_cnt)

    norm2 = jnp.sum(kmat * kmat, axis=0, keepdims=True)
    kn = (kmat * jax.lax.rsqrt(jnp.maximum(norm2, 1e-30))).astype(jnp.bfloat16)
    cos = jax.lax.dot_general(x_tile, kn, (((1,), (0,)), ((), ())),
                              preferred_element_type=jnp.float32)
    cos = jnp.clip(cos, -1.0, 1.0)
    col = jax.lax.broadcasted_iota(jnp.int32, cos.shape, 1)
    is_tgt = col == t_col
    logits = jnp.where(is_tgt, cos - _BETA, cos) * _ALPHA
    logits = jnp.where(col < n_classes, logits, -1e30)
    m = jnp.max(logits, axis=1, keepdims=True)
    lse = m + jnp.log(jnp.sum(jnp.exp(logits - m), axis=1, keepdims=True))
    tgt_logit = jnp.sum(jnp.where(is_tgt, logits, 0.0), axis=1, keepdims=True)
    loss_part = jnp.sum(lse - tgt_logit)
    pred = jnp.min(jnp.where(logits == m, col, jnp.int32(2**30)),
                   axis=1, keepdims=True)
    prec_part = jnp.sum((pred == t_col).astype(jnp.float32))

    row = jax.lax.broadcasted_iota(jnp.int32, acc_ref.shape, 0)
    partials = jnp.where(
        row == 0, loss_part,
        jnp.where(row == 1, prec_part,
                  jnp.where(row == 2, pos_part,
                            jnp.where(row == 3, neg_part, 0.0))))

    @pl.when(i == 0)
    def _init():
        acc_ref[...] = partials

    @pl.when(i != 0)
    def _accum():
        acc_ref[...] += partials


def kernel(inputs, targets, kmat):
    n, d = inputs.shape
    c = kmat.shape[1]
    cp = (c + 127) // 128 * 128
    tile = 256
    grid = n // tile

    x_bf = inputs.astype(jnp.bfloat16)
    k_pad = jnp.pad(kmat, ((0, 0), (0, cp - c)))
    t_col = targets.reshape(n, 1)
    t_row = targets.reshape(1, n)

    acc = pl.pallas_call(
        functools.partial(_fused_loss_kernel, n_classes=c),
        grid=(grid,),
        in_specs=[
            pl.BlockSpec((tile, d), lambda i: (i, 0)),
            pl.BlockSpec((n, d), lambda i: (0, 0)),
            pl.BlockSpec((d, cp), lambda i: (0, 0)),
            pl.BlockSpec((tile, 1), lambda i: (i, 0)),
            pl.BlockSpec((1, n), lambda i: (0, 0)),
        ],
        out_specs=pl.BlockSpec((8, 128), lambda i: (0, 0)),
        out_shape=jax.ShapeDtypeStruct((8, 128), jnp.float32),
        compiler_params=pltpu.CompilerParams(
            dimension_semantics=("arbitrary",)),
    )(x_bf, x_bf, k_pad, t_col, t_row)

    nf = jnp.float32(n)
    return (acc[0, 0] / nf, acc[1, 0] / nf, acc[2, 0] / nf, acc[3, 0] / nf)


# speedup vs baseline: 1.7345x; 1.1100x over previous
import functools

import jax
import jax.numpy as jnp
from jax.experimental import pallas as pl
from jax.experimental.pallas import tpu as pltpu

_ALPHA = 10.0
_BETA = 2.0


def _fused_loss_kernel(x_tile_ref, x_full_ref, k_ref, t_col_ref, t_row_ref,
                       acc_ref, *, n_classes):
    i = pl.program_id(0)
    x_tile = x_tile_ref[...]
    x_full = x_full_ref[...]
    kmat = k_ref[...]
    t_col = t_col_ref[...]
    t_row = t_row_ref[...]

    sim = jax.lax.dot_general(x_tile, x_full, (((1,), (1,)), ((), ())),
                              preferred_element_type=jnp.float32)
    same = t_col == t_row
    pos_mask = jnp.logical_and(same, sim < 1.0)
    pos_cnt = jnp.sum(pos_mask.astype(jnp.float32), axis=1, keepdims=True)
    same_cnt = jnp.sum(same.astype(jnp.float32), axis=1, keepdims=True)
    neg_cnt = jnp.float32(sim.shape[1]) - same_cnt
    pos_sum = jnp.sum(jnp.where(pos_mask, sim, 0.0), axis=1, keepdims=True)
    tot_sum = jnp.sum(sim, axis=1, keepdims=True)
    same_sum = jnp.sum(jnp.where(same, sim, 0.0), axis=1, keepdims=True)
    neg_sum = tot_sum - same_sum
    pos_part = jnp.sum(pos_sum / pos_cnt)
    neg_part = jnp.sum(neg_sum / neg_cnt)

    norm2 = jnp.sum(kmat * kmat, axis=0, keepdims=True)
    kn = (kmat * jax.lax.rsqrt(jnp.maximum(norm2, 1e-30))).astype(jnp.bfloat16)
    cos = jax.lax.dot_general(x_tile, kn, (((1,), (0,)), ((), ())),
                              preferred_element_type=jnp.float32)
    cos = jnp.clip(cos, -1.0, 1.0)
    col = jax.lax.broadcasted_iota(jnp.int32, cos.shape, 1)
    is_tgt = col == t_col
    logits = jnp.where(is_tgt, cos - _BETA, cos) * _ALPHA
    logits = jnp.where(col < n_classes, logits, -1e30)
    m = jnp.max(logits, axis=1, keepdims=True)
    lse = m + jnp.log(jnp.sum(jnp.exp(logits - m), axis=1, keepdims=True))
    tgt_logit = jnp.sum(jnp.where(is_tgt, logits, 0.0), axis=1, keepdims=True)
    loss_part = jnp.sum(lse - tgt_logit)
    pred = jnp.min(jnp.where(logits == m, col, jnp.int32(2**30)),
                   axis=1, keepdims=True)
    prec_part = jnp.sum((pred == t_col).astype(jnp.float32))

    row = jax.lax.broadcasted_iota(jnp.int32, acc_ref.shape, 0)
    partials = jnp.where(
        row == 0, loss_part,
        jnp.where(row == 1, prec_part,
                  jnp.where(row == 2, pos_part,
                            jnp.where(row == 3, neg_part, 0.0))))

    @pl.when(i == 0)
    def _init():
        acc_ref[...] = partials

    @pl.when(i != 0)
    def _accum():
        acc_ref[...] += partials


def kernel(inputs, targets, kmat):
    n, d = inputs.shape
    c = kmat.shape[1]
    cp = (c + 127) // 128 * 128
    tile = 512
    grid = n // tile

    x_bf = inputs.astype(jnp.bfloat16)
    k_pad = jnp.pad(kmat, ((0, 0), (0, cp - c)))
    t_col = targets.reshape(n, 1)
    t_row = targets.reshape(1, n)

    acc = pl.pallas_call(
        functools.partial(_fused_loss_kernel, n_classes=c),
        grid=(grid,),
        in_specs=[
            pl.BlockSpec((tile, d), lambda i: (i, 0)),
            pl.BlockSpec((n, d), lambda i: (0, 0)),
            pl.BlockSpec((d, cp), lambda i: (0, 0)),
            pl.BlockSpec((tile, 1), lambda i: (i, 0)),
            pl.BlockSpec((1, n), lambda i: (0, 0)),
        ],
        out_specs=pl.BlockSpec((8, 128), lambda i: (0, 0)),
        out_shape=jax.ShapeDtypeStruct((8, 128), jnp.float32),
        compiler_params=pltpu.CompilerParams(
            dimension_semantics=("arbitrary",)),
    )(x_bf, x_bf, k_pad, t_col, t_row)

    nf = jnp.float32(n)
    return (acc[0, 0] / nf, acc[1, 0] / nf, acc[2, 0] / nf, acc[3, 0] / nf)
